# baseline (speedup 1.0000x reference)
import functools

import jax
import jax.numpy as jnp
from jax import lax
from jax.experimental import pallas as pl
from jax.experimental.pallas import tpu as pltpu
from jax.experimental.pallas import tpu_sc as plsc

G, S, DIM, E, TOPN = 10, 2048, 2560, 8, 2
INTER = 6912
C = 384
M = G * C
MC = 768
NM = M // MC
HB = 384
NH = INTER // HB
NSLOT = E * M


def _pack_pair(lo_bf, hi_bf):
    li = lax.bitcast_convert_type(lo_bf.astype(jnp.float32), jnp.int32)
    ri = lax.bitcast_convert_type(hi_bf.astype(jnp.float32), jnp.int32)
    return lax.shift_right_logical(li, 16) | (ri & jnp.int32(-65536))


def _unpack_pair(x32):
    lo = lax.bitcast_convert_type(lax.shift_left(x32, 16), jnp.float32)
    hi = lax.bitcast_convert_type(x32 & jnp.int32(-65536), jnp.float32)
    return lo.astype(jnp.bfloat16), hi.astype(jnp.bfloat16)


def _logits_body(x_ref, w_ref, o_ref, x32_ref):
    xb = x_ref[...]
    o_ref[...] = jnp.dot(xb.astype(jnp.float32), w_ref[...],
                         preferred_element_type=jnp.float32)
    x32_ref[...] = _pack_pair(xb[:, :DIM // 2], xb[:, DIM // 2:])


def _router_logits(xf, router_gate):
    wpad = jnp.zeros((DIM, 128), jnp.float32).at[:, :E].set(router_gate)
    out, x32 = pl.pallas_call(
        _logits_body,
        grid=(G,),
        in_specs=[pl.BlockSpec((S, DIM), lambda i: (i, 0)),
                  pl.BlockSpec((DIM, 128), lambda i: (0, 0))],
        out_specs=[pl.BlockSpec((S, 128), lambda i: (i, 0)),
                   pl.BlockSpec((S, DIM // 2), lambda i: (i, 0))],
        out_shape=[jax.ShapeDtypeStruct((G * S, 128), jnp.float32),
                   jax.ShapeDtypeStruct((G * S, DIM // 2), jnp.int32)],
    )(xf, wpad)
    return out[:, :E], x32


def _mlp_body(x_ref, wg_ref, wi_ref, wo_ref, o_ref, acc_ref, xl_ref, xr_ref):
    h = pl.program_id(2)

    @pl.when(h == 0)
    def _():
        xl, xr = _unpack_pair(x_ref[0])
        xl_ref[...] = xl
        xr_ref[...] = xr

    xl = xl_ref[...]
    xr = xr_ref[...]
    D2 = DIM // 2
    wg = wg_ref[0]
    wi = wi_ref[0]
    h1 = (jnp.dot(xl, wg[:D2], preferred_element_type=jnp.float32)
          + jnp.dot(xr, wg[D2:], preferred_element_type=jnp.float32))
    h0 = (jnp.dot(xl, wi[:D2], preferred_element_type=jnp.float32)
          + jnp.dot(xr, wi[D2:], preferred_element_type=jnp.float32))
    hh = jax.nn.silu(h1.astype(jnp.bfloat16)) * h0.astype(jnp.bfloat16)
    y = jnp.dot(hh, wo_ref[0], preferred_element_type=jnp.float32)

    @pl.when(h == 0)
    def _():
        acc_ref[...] = y

    @pl.when(h > 0)
    def _():
        acc_ref[...] += y

    @pl.when(h == NH - 1)
    def _():
        yb = acc_ref[...].astype(jnp.bfloat16)
        o_ref[0] = _pack_pair(yb[:, :D2], yb[:, D2:])


def _expert_mlp(Xe_i32, wg, wi, wo):
    return pl.pallas_call(
        _mlp_body,
        grid=(E, NM, NH),
        in_specs=[
            pl.BlockSpec((1, MC, DIM // 2), lambda e, m, h: (e, m, 0)),
            pl.BlockSpec((1, DIM, HB), lambda e, m, h: (e, 0, h)),
            pl.BlockSpec((1, DIM, HB), lambda e, m, h: (e, 0, h)),
            pl.BlockSpec((1, HB, DIM), lambda e, m, h: (e, h, 0)),
        ],
        out_specs=pl.BlockSpec((1, MC, DIM // 2), lambda e, m, h: (e, m, 0)),
        out_shape=jax.ShapeDtypeStruct((E, M, DIM // 2), jnp.int32),
        scratch_shapes=[pltpu.VMEM((MC, DIM), jnp.float32),
                        pltpu.VMEM((MC, DIM // 2), jnp.bfloat16),
                        pltpu.VMEM((MC, DIM // 2), jnp.bfloat16)],
        compiler_params=pltpu.CompilerParams(
            dimension_semantics=("parallel", "parallel", "arbitrary")),
    )(Xe_i32, wg, wi, wo)


_INFO = plsc.get_sparse_core_info()
NC, NS = _INFO.num_cores, _INFO.num_subcores
NW = NC * NS
TB = 16
NBLK = S // TB


def _wid():
    return lax.axis_index("s") * NC + lax.axis_index("c")


def _dyn_splat_i(vec, lane):
    dnums = lax.GatherDimensionNumbers(
        offset_dims=(), collapsed_slice_dims=(0,), start_index_map=(0,))
    idx = jnp.full((16, 1), lane, jnp.int32)
    return lax.gather(vec, idx, dnums, slice_sizes=(1,),
                      mode=lax.GatherScatterMode.PROMISE_IN_BOUNDS)


def _route_body(lg_hbm, stok_hbm, cidx_hbm, cgate_hbm,
                lg, stok, cbuf, gbuf):
    g = _wid()
    zi = jnp.zeros((16,), jnp.int32)

    @pl.when(g < G)
    def _():
        pltpu.sync_copy(lg_hbm.at[pl.ds(g * S * E, S * E)], lg)
        def _z(i, _):
            stok[pl.ds(i * 16, 16)] = zi
            return 0
        lax.fori_loop(0, (E * C) // 16, _z, 0)

        iot = lax.iota(jnp.int32, 16)

        def blk(b, cnt):
            t0 = b * TB
            vs = []
            for e in range(E):
                vs.append(plsc.load_gather(lg, [(t0 + iot) * E + e]))
            max1 = vs[0]
            for e in range(1, E):
                max1 = jnp.maximum(max1, vs[e])
            arg1 = zi + E
            for e in range(E - 1, -1, -1):
                arg1 = jnp.where(vs[e] == max1, e, arg1)
            max2 = None
            for e in range(E):
                m = jnp.where(arg1 == e, -jnp.inf, vs[e])
                max2 = m if max2 is None else jnp.maximum(max2, m)
            arg2 = zi + E
            for e in range(E - 1, -1, -1):
                arg2 = jnp.where(
                    (arg1 != e) & (jnp.where(arg1 == e, -jnp.inf, vs[e]) == max2),
                    e, arg2)
            bexp = jnp.exp(max2 - max1)
            g1 = 1.0 / (1.0 + bexp)
            g2 = bexp / (1.0 + bexp)

            pos1 = zi
            pos2 = zi
            ncnt = []
            for e in range(E):
                m1 = arg1 == e
                m2 = arg2 == e
                i1 = jnp.where(m1, 1, 0)
                i2 = jnp.where(m2, 1, 0)
                c1 = plsc.cumsum(i1)
                c2 = plsc.cumsum(i2)
                base = cnt[e]
                pos1 = jnp.where(m1, base + c1 - 1 + c2 - i2, pos1)
                pos2 = jnp.where(m2, base + c1 + c2 - 1, pos2)
                ncnt.append(base + _dyn_splat_i(c1, 15) + _dyn_splat_i(c2, 15))
            cnt = tuple(ncnt)
            ok1 = pos1 < C
            ok2 = pos2 < C
            tok = g * S + t0 + iot
            plsc.store_scatter(stok, [arg1 * C + pos1], tok, mask=ok1)
            plsc.store_scatter(stok, [arg2 * C + pos2], tok, mask=ok2)
            slot1 = (arg1 * G + g) * C + pos1
            slot2 = (arg2 * G + g) * C + pos2
            ai = 2 * (t0 + iot)
            plsc.store_scatter(cbuf, [ai], jnp.where(ok1, slot1, 0))
            plsc.store_scatter(cbuf, [ai + 1], jnp.where(ok2, slot2, 0))
            plsc.store_scatter(gbuf, [ai], jnp.where(ok1, g1, 0.0))
            plsc.store_scatter(gbuf, [ai + 1], jnp.where(ok2, g2, 0.0))
            return cnt

        zv = jnp.zeros((16,), jnp.int32)
        lax.fori_loop(0, NBLK, blk, tuple(zv for _ in range(E)),
                      unroll=False)

        for e in range(E):
            pltpu.sync_copy(stok.at[pl.ds(e * C, C)],
                            stok_hbm.at[pl.ds((e * G + g) * C, C)])
        pltpu.sync_copy(cbuf, cidx_hbm.at[pl.ds(g * S * TOPN, S * TOPN)])
        pltpu.sync_copy(gbuf, cgate_hbm.at[pl.ds(g * S * TOPN, S * TOPN)])


def _sc_route(logits8):
    mesh = plsc.VectorSubcoreMesh(core_axis_name="c", subcore_axis_name="s")
    fn = pl.kernel(
        _route_body,
        out_type=[jax.ShapeDtypeStruct((NSLOT,), jnp.int32),
                  jax.ShapeDtypeStruct((G * S * TOPN,), jnp.int32),
                  jax.ShapeDtypeStruct((G * S * TOPN,), jnp.float32)],
        mesh=mesh,
        scratch_types=[pltpu.VMEM((S * E,), jnp.float32),
                       pltpu.VMEM((E * C,), jnp.int32),
                       pltpu.VMEM((S * TOPN,), jnp.int32),
                       pltpu.VMEM((S * TOPN,), jnp.float32)],
        compiler_params=pltpu.CompilerParams(needs_layout_passes=False),
    )
    return fn(logits8.reshape(G * S * E))


_GK = 48
_GR = NSLOT // NW
_GN = _GR // _GK


def _gather_body(xf_hbm, st_hbm, xe_hbm, idxb, rows0, rows1, sem0, sem1):
    w = _wid()
    base = w * _GR
    pltpu.sync_copy(st_hbm.at[pl.ds(base, _GR)], idxb)

    def start(i, rows, sem):
        pltpu.async_copy(xf_hbm.at[idxb.at[pl.ds(i * _GK, _GK)]], rows, sem)

    start(0, rows0, sem0)

    def step(i, _):
        even = lax.rem(i, 2) == 0

        @pl.when(i + 1 < _GN)
        def _():
            @pl.when(even)
            def _():
                start(i + 1, rows1, sem1)

            @pl.when(jnp.logical_not(even))
            def _():
                start(i + 1, rows0, sem0)

        @pl.when(even)
        def _():
            pltpu.make_async_copy(xf_hbm.at[idxb.at[pl.ds(i * _GK, _GK)]],
                                  rows0, sem0).wait()
            pltpu.sync_copy(rows0, xe_hbm.at[pl.ds(base + i * _GK, _GK)])

        @pl.when(jnp.logical_not(even))
        def _():
            pltpu.make_async_copy(xf_hbm.at[idxb.at[pl.ds(i * _GK, _GK)]],
                                  rows1, sem1).wait()
            pltpu.sync_copy(rows1, xe_hbm.at[pl.ds(base + i * _GK, _GK)])
        return 0

    lax.fori_loop(0, _GN, step, 0)


def _sc_gather(xf_i32, slot_token):
    mesh = plsc.VectorSubcoreMesh(core_axis_name="c", subcore_axis_name="s")
    fn = pl.kernel(
        _gather_body,
        out_type=jax.ShapeDtypeStruct((NSLOT, DIM // 2), jnp.int32),
        mesh=mesh,
        scratch_types=[pltpu.VMEM((_GR,), jnp.int32),
                       pltpu.VMEM((_GK, DIM // 2), jnp.int32),
                       pltpu.VMEM((_GK, DIM // 2), jnp.int32),
                       pltpu.SemaphoreType.DMA,
                       pltpu.SemaphoreType.DMA],
        compiler_params=pltpu.CompilerParams(needs_layout_passes=False),
    )
    return fn(xf_i32, slot_token)


_CT = 8
_CW = (G * S) // NW
_CN = _CW // _CT


def _dyn_splat(vec, lane):
    dnums = lax.GatherDimensionNumbers(
        offset_dims=(), collapsed_slice_dims=(0,), start_index_map=(0,))
    idx = jnp.full((16, 1), lane, jnp.int32)
    return lax.gather(vec, idx, dnums, slice_sizes=(1,),
                      mode=lax.GatherScatterMode.PROMISE_IN_BOUNDS)


def _combine_body(y_hbm, ci_hbm, cg_hbm, out_hbm, cib, cgb, rows0, rows1,
                  obuf, sem0, sem1):
    w = _wid()
    tbase = w * _CW
    pltpu.sync_copy(ci_hbm.at[pl.ds(tbase * 2, _CW * 2)], cib)
    pltpu.sync_copy(cg_hbm.at[pl.ds(tbase * 2, _CW * 2)], cgb)

    def start(i, rows, sem):
        pltpu.async_copy(y_hbm.at[cib.at[pl.ds(i * 2 * _CT, 2 * _CT)]], rows, sem)

    start(0, rows0, sem0)

    def compute(i, rows):
        gv = cgb[pl.ds(i * 2 * _CT, 2 * _CT)]
        for t in range(_CT):
            ga = _dyn_splat(gv, 2 * t)
            gb = _dyn_splat(gv, 2 * t + 1)
            gab = plsc.pack(ga, ga, format=plsc.PackFormat.INTERLEAVED)
            gbb = plsc.pack(gb, gb, format=plsc.PackFormat.INTERLEAVED)

            def chunk(j, _):
                a = plsc.bitcast(rows[2 * t, pl.ds(j * 16, 16)], jnp.bfloat16)
                bb = plsc.bitcast(rows[2 * t + 1, pl.ds(j * 16, 16)],
                                  jnp.bfloat16)
                obuf[t, pl.ds(j * 16, 16)] = plsc.bitcast(a * gab + bb * gbb,
                                                          jnp.int32)
                return 0

            lax.fori_loop(0, DIM // 32, chunk, 0, unroll=8)
        pltpu.sync_copy(obuf, out_hbm.at[pl.ds(tbase + i * _CT, _CT)])

    def step(i, _):
        even = lax.rem(i, 2) == 0

        @pl.when(i + 1 < _CN)
        def _():
            @pl.when(even)
            def _():
                start(i + 1, rows1, sem1)

            @pl.when(jnp.logical_not(even))
            def _():
                start(i + 1, rows0, sem0)

        @pl.when(even)
        def _():
            pltpu.make_async_copy(y_hbm.at[cib.at[pl.ds(i * 2 * _CT, 2 * _CT)]],
                                  rows0, sem0).wait()
            compute(i, rows0)

        @pl.when(jnp.logical_not(even))
        def _():
            pltpu.make_async_copy(y_hbm.at[cib.at[pl.ds(i * 2 * _CT, 2 * _CT)]],
                                  rows1, sem1).wait()
            compute(i, rows1)
        return 0

    lax.fori_loop(0, _CN, step, 0)


def _sc_combine(Yf_i32, cidx, cgate):
    mesh = plsc.VectorSubcoreMesh(core_axis_name="c", subcore_axis_name="s")
    fn = pl.kernel(
        _combine_body,
        out_type=jax.ShapeDtypeStruct((G * S, DIM // 2), jnp.int32),
        mesh=mesh,
        scratch_types=[pltpu.VMEM((_CW * 2,), jnp.int32),
                       pltpu.VMEM((_CW * 2,), jnp.float32),
                       pltpu.VMEM((2 * _CT, DIM // 2), jnp.int32),
                       pltpu.VMEM((2 * _CT, DIM // 2), jnp.int32),
                       pltpu.VMEM((_CT, DIM // 2), jnp.int32),
                       pltpu.SemaphoreType.DMA,
                       pltpu.SemaphoreType.DMA],
        compiler_params=pltpu.CompilerParams(needs_layout_passes=False),
    )
    return fn(Yf_i32, cidx, cgate)


def _unpack_body(x_ref, o_ref):
    lo, hi = _unpack_pair(x_ref[...])
    o_ref[...] = jnp.concatenate([lo, hi], axis=-1)


def _unpack_out(o32):
    return pl.pallas_call(
        _unpack_body,
        grid=(G,),
        in_specs=[pl.BlockSpec((S, DIM // 2), lambda i: (i, 0))],
        out_specs=pl.BlockSpec((S, DIM), lambda i: (i, 0)),
        out_shape=jax.ShapeDtypeStruct((G * S, DIM), jnp.bfloat16),
    )(o32)


def kernel(inputs, wi_gate_0, wi_0, wo_0, router_gate):
    bf = inputs.dtype
    xf = inputs.reshape(G * S, DIM)
    logits, x32 = _router_logits(xf, router_gate)
    slot_token, cidx, cgate = _sc_route(logits)

    Xe_i32 = _sc_gather(x32, slot_token)
    Y_i32 = _expert_mlp(Xe_i32.reshape(E, M, DIM // 2), wi_gate_0, wi_0, wo_0)
    o32 = _sc_combine(Y_i32.reshape(NSLOT, DIM // 2), cidx, cgate)
    out = _unpack_out(o32)
    return out.reshape(inputs.shape)


# speedup vs baseline: 1.1822x; 1.1822x over previous
import functools

import jax
import jax.numpy as jnp
from jax import lax
from jax.experimental import pallas as pl
from jax.experimental.pallas import tpu as pltpu
from jax.experimental.pallas import tpu_sc as plsc

G, S, DIM, E, TOPN = 10, 2048, 2560, 8, 2
INTER = 6912
C = 384
M = G * C
MC = 768
NM = M // MC
HB = 768
NH = INTER // HB
NSLOT = E * M


def _pack_pair(lo_bf, hi_bf):
    li = lax.bitcast_convert_type(lo_bf.astype(jnp.float32), jnp.int32)
    ri = lax.bitcast_convert_type(hi_bf.astype(jnp.float32), jnp.int32)
    return lax.shift_right_logical(li, 16) | (ri & jnp.int32(-65536))


def _unpack_pair(x32):
    lo = lax.bitcast_convert_type(lax.shift_left(x32, 16), jnp.float32)
    hi = lax.bitcast_convert_type(x32 & jnp.int32(-65536), jnp.float32)
    return lo.astype(jnp.bfloat16), hi.astype(jnp.bfloat16)


def _logits_body(x_ref, w_ref, o_ref, x32_ref):
    xb = x_ref[...]
    o_ref[...] = jnp.dot(xb.astype(jnp.float32), w_ref[...],
                         preferred_element_type=jnp.float32)
    x32_ref[...] = _pack_pair(xb[:, :DIM // 2], xb[:, DIM // 2:])


def _router_logits(xf, router_gate):
    wpad = jnp.zeros((DIM, 128), jnp.float32).at[:, :E].set(router_gate)
    out, x32 = pl.pallas_call(
        _logits_body,
        grid=(G,),
        in_specs=[pl.BlockSpec((S, DIM), lambda i: (i, 0)),
                  pl.BlockSpec((DIM, 128), lambda i: (0, 0))],
        out_specs=[pl.BlockSpec((S, 128), lambda i: (i, 0)),
                   pl.BlockSpec((S, DIM // 2), lambda i: (i, 0))],
        out_shape=[jax.ShapeDtypeStruct((G * S, 128), jnp.float32),
                   jax.ShapeDtypeStruct((G * S, DIM // 2), jnp.int32)],
    )(xf, wpad)
    return out[:, :E], x32


def _mlp_body(x_ref, wg_ref, wi_ref, wo_ref, o_ref, acc_ref, xl_ref, xr_ref):
    h = pl.program_id(2)

    @pl.when(h == 0)
    def _():
        xl, xr = _unpack_pair(x_ref[0])
        xl_ref[...] = xl
        xr_ref[...] = xr

    xl = xl_ref[...]
    xr = xr_ref[...]
    D2 = DIM // 2
    wg = wg_ref[0]
    wi = wi_ref[0]
    h1 = (jnp.dot(xl, wg[:D2], preferred_element_type=jnp.float32)
          + jnp.dot(xr, wg[D2:], preferred_element_type=jnp.float32))
    h0 = (jnp.dot(xl, wi[:D2], preferred_element_type=jnp.float32)
          + jnp.dot(xr, wi[D2:], preferred_element_type=jnp.float32))
    hh = jax.nn.silu(h1.astype(jnp.bfloat16)) * h0.astype(jnp.bfloat16)
    y = jnp.dot(hh, wo_ref[0], preferred_element_type=jnp.float32)

    @pl.when(h == 0)
    def _():
        acc_ref[...] = y

    @pl.when(h > 0)
    def _():
        acc_ref[...] += y

    @pl.when(h == NH - 1)
    def _():
        yb = acc_ref[...].astype(jnp.bfloat16)
        o_ref[0] = _pack_pair(yb[:, :D2], yb[:, D2:])


def _expert_mlp(Xe_i32, wg, wi, wo):
    return pl.pallas_call(
        _mlp_body,
        grid=(E, NM, NH),
        in_specs=[
            pl.BlockSpec((1, MC, DIM // 2), lambda e, m, h: (e, m, 0)),
            pl.BlockSpec((1, DIM, HB), lambda e, m, h: (e, 0, h)),
            pl.BlockSpec((1, DIM, HB), lambda e, m, h: (e, 0, h)),
            pl.BlockSpec((1, HB, DIM), lambda e, m, h: (e, h, 0)),
        ],
        out_specs=pl.BlockSpec((1, MC, DIM // 2), lambda e, m, h: (e, m, 0)),
        out_shape=jax.ShapeDtypeStruct((E, M, DIM // 2), jnp.int32),
        scratch_shapes=[pltpu.VMEM((MC, DIM), jnp.float32),
                        pltpu.VMEM((MC, DIM // 2), jnp.bfloat16),
                        pltpu.VMEM((MC, DIM // 2), jnp.bfloat16)],
        compiler_params=pltpu.CompilerParams(
            dimension_semantics=("parallel", "parallel", "arbitrary")),
    )(Xe_i32, wg, wi, wo)


_INFO = plsc.get_sparse_core_info()
NC, NS = _INFO.num_cores, _INFO.num_subcores
NW = NC * NS
TB = 16
NBLK = S // TB


def _wid():
    return lax.axis_index("s") * NC + lax.axis_index("c")


def _dyn_splat_i(vec, lane):
    dnums = lax.GatherDimensionNumbers(
        offset_dims=(), collapsed_slice_dims=(0,), start_index_map=(0,))
    idx = jnp.full((16, 1), lane, jnp.int32)
    return lax.gather(vec, idx, dnums, slice_sizes=(1,),
                      mode=lax.GatherScatterMode.PROMISE_IN_BOUNDS)


def _route_body(lg_hbm, stok_hbm, cidx_hbm, cgate_hbm,
                lg, stok, cbuf, gbuf):
    g = _wid()
    zi = jnp.zeros((16,), jnp.int32)

    @pl.when(g < G)
    def _():
        pltpu.sync_copy(lg_hbm.at[pl.ds(g * S * E, S * E)], lg)
        def _z(i, _):
            stok[pl.ds(i * 16, 16)] = zi
            return 0
        lax.fori_loop(0, (E * C) // 16, _z, 0)

        iot = lax.iota(jnp.int32, 16)

        def blk(b, cnt):
            t0 = b * TB
            vs = []
            for e in range(E):
                vs.append(plsc.load_gather(lg, [(t0 + iot) * E + e]))
            max1 = vs[0]
            for e in range(1, E):
                max1 = jnp.maximum(max1, vs[e])
            arg1 = zi + E
            for e in range(E - 1, -1, -1):
                arg1 = jnp.where(vs[e] == max1, e, arg1)
            max2 = None
            for e in range(E):
                m = jnp.where(arg1 == e, -jnp.inf, vs[e])
                max2 = m if max2 is None else jnp.maximum(max2, m)
            arg2 = zi + E
            for e in range(E - 1, -1, -1):
                arg2 = jnp.where(
                    (arg1 != e) & (jnp.where(arg1 == e, -jnp.inf, vs[e]) == max2),
                    e, arg2)
            bexp = jnp.exp(max2 - max1)
            g1 = 1.0 / (1.0 + bexp)
            g2 = bexp / (1.0 + bexp)

            pos1 = zi
            pos2 = zi
            ncnt = []
            for e in range(E):
                m1 = arg1 == e
                m2 = arg2 == e
                i1 = jnp.where(m1, 1, 0)
                i2 = jnp.where(m2, 1, 0)
                c1 = plsc.cumsum(i1)
                c2 = plsc.cumsum(i2)
                base = cnt[e]
                pos1 = jnp.where(m1, base + c1 - 1 + c2 - i2, pos1)
                pos2 = jnp.where(m2, base + c1 + c2 - 1, pos2)
                ncnt.append(base + _dyn_splat_i(c1, 15) + _dyn_splat_i(c2, 15))
            cnt = tuple(ncnt)
            ok1 = pos1 < C
            ok2 = pos2 < C
            tok = g * S + t0 + iot
            plsc.store_scatter(stok, [arg1 * C + pos1], tok, mask=ok1)
            plsc.store_scatter(stok, [arg2 * C + pos2], tok, mask=ok2)
            slot1 = (arg1 * G + g) * C + pos1
            slot2 = (arg2 * G + g) * C + pos2
            ai = 2 * (t0 + iot)
            plsc.store_scatter(cbuf, [ai], jnp.where(ok1, slot1, 0))
            plsc.store_scatter(cbuf, [ai + 1], jnp.where(ok2, slot2, 0))
            plsc.store_scatter(gbuf, [ai], jnp.where(ok1, g1, 0.0))
            plsc.store_scatter(gbuf, [ai + 1], jnp.where(ok2, g2, 0.0))
            return cnt

        zv = jnp.zeros((16,), jnp.int32)
        lax.fori_loop(0, NBLK, blk, tuple(zv for _ in range(E)),
                      unroll=False)

        for e in range(E):
            pltpu.sync_copy(stok.at[pl.ds(e * C, C)],
                            stok_hbm.at[pl.ds((e * G + g) * C, C)])
        pltpu.sync_copy(cbuf, cidx_hbm.at[pl.ds(g * S * TOPN, S * TOPN)])
        pltpu.sync_copy(gbuf, cgate_hbm.at[pl.ds(g * S * TOPN, S * TOPN)])


def _sc_route(logits8):
    mesh = plsc.VectorSubcoreMesh(core_axis_name="c", subcore_axis_name="s")
    fn = pl.kernel(
        _route_body,
        out_type=[jax.ShapeDtypeStruct((NSLOT,), jnp.int32),
                  jax.ShapeDtypeStruct((G * S * TOPN,), jnp.int32),
                  jax.ShapeDtypeStruct((G * S * TOPN,), jnp.float32)],
        mesh=mesh,
        scratch_types=[pltpu.VMEM((S * E,), jnp.float32),
                       pltpu.VMEM((E * C,), jnp.int32),
                       pltpu.VMEM((S * TOPN,), jnp.int32),
                       pltpu.VMEM((S * TOPN,), jnp.float32)],
        compiler_params=pltpu.CompilerParams(needs_layout_passes=False),
    )
    return fn(logits8.reshape(G * S * E))


_GK = 48
_GR = NSLOT // NW
_GN = _GR // _GK


def _gather_body(xf_hbm, st_hbm, xe_hbm, idxb, rows0, rows1, sem0, sem1):
    w = _wid()
    base = w * _GR
    pltpu.sync_copy(st_hbm.at[pl.ds(base, _GR)], idxb)

    def start(i, rows, sem):
        pltpu.async_copy(xf_hbm.at[idxb.at[pl.ds(i * _GK, _GK)]], rows, sem)

    start(0, rows0, sem0)

    def step(i, _):
        even = lax.rem(i, 2) == 0

        @pl.when(i + 1 < _GN)
        def _():
            @pl.when(even)
            def _():
                start(i + 1, rows1, sem1)

            @pl.when(jnp.logical_not(even))
            def _():
                start(i + 1, rows0, sem0)

        @pl.when(even)
        def _():
            pltpu.make_async_copy(xf_hbm.at[idxb.at[pl.ds(i * _GK, _GK)]],
                                  rows0, sem0).wait()
            pltpu.sync_copy(rows0, xe_hbm.at[pl.ds(base + i * _GK, _GK)])

        @pl.when(jnp.logical_not(even))
        def _():
            pltpu.make_async_copy(xf_hbm.at[idxb.at[pl.ds(i * _GK, _GK)]],
                                  rows1, sem1).wait()
            pltpu.sync_copy(rows1, xe_hbm.at[pl.ds(base + i * _GK, _GK)])
        return 0

    lax.fori_loop(0, _GN, step, 0)


def _sc_gather(xf_i32, slot_token):
    mesh = plsc.VectorSubcoreMesh(core_axis_name="c", subcore_axis_name="s")
    fn = pl.kernel(
        _gather_body,
        out_type=jax.ShapeDtypeStruct((NSLOT, DIM // 2), jnp.int32),
        mesh=mesh,
        scratch_types=[pltpu.VMEM((_GR,), jnp.int32),
                       pltpu.VMEM((_GK, DIM // 2), jnp.int32),
                       pltpu.VMEM((_GK, DIM // 2), jnp.int32),
                       pltpu.SemaphoreType.DMA,
                       pltpu.SemaphoreType.DMA],
        compiler_params=pltpu.CompilerParams(needs_layout_passes=False),
    )
    return fn(xf_i32, slot_token)


_CT = 8
_CW = (G * S) // NW
_CN = _CW // _CT


def _dyn_splat(vec, lane):
    dnums = lax.GatherDimensionNumbers(
        offset_dims=(), collapsed_slice_dims=(0,), start_index_map=(0,))
    idx = jnp.full((16, 1), lane, jnp.int32)
    return lax.gather(vec, idx, dnums, slice_sizes=(1,),
                      mode=lax.GatherScatterMode.PROMISE_IN_BOUNDS)


def _combine_body(y_hbm, ci_hbm, cg_hbm, out_hbm, cib, cgb, rows0, rows1,
                  obuf, sem0, sem1):
    w = _wid()
    tbase = w * _CW
    pltpu.sync_copy(ci_hbm.at[pl.ds(tbase * 2, _CW * 2)], cib)
    pltpu.sync_copy(cg_hbm.at[pl.ds(tbase * 2, _CW * 2)], cgb)

    def start(i, rows, sem):
        pltpu.async_copy(y_hbm.at[cib.at[pl.ds(i * 2 * _CT, 2 * _CT)]], rows, sem)

    start(0, rows0, sem0)

    def compute(i, rows):
        gv = cgb[pl.ds(i * 2 * _CT, 2 * _CT)]
        for t in range(_CT):
            ga = _dyn_splat(gv, 2 * t)
            gb = _dyn_splat(gv, 2 * t + 1)
            gab = plsc.pack(ga, ga, format=plsc.PackFormat.INTERLEAVED)
            gbb = plsc.pack(gb, gb, format=plsc.PackFormat.INTERLEAVED)

            def chunk(j, _):
                a = plsc.bitcast(rows[2 * t, pl.ds(j * 16, 16)], jnp.bfloat16)
                bb = plsc.bitcast(rows[2 * t + 1, pl.ds(j * 16, 16)],
                                  jnp.bfloat16)
                obuf[t, pl.ds(j * 16, 16)] = plsc.bitcast(a * gab + bb * gbb,
                                                          jnp.int32)
                return 0

            lax.fori_loop(0, DIM // 32, chunk, 0, unroll=8)
        pltpu.sync_copy(obuf, out_hbm.at[pl.ds(tbase + i * _CT, _CT)])

    def step(i, _):
        even = lax.rem(i, 2) == 0

        @pl.when(i + 1 < _CN)
        def _():
            @pl.when(even)
            def _():
                start(i + 1, rows1, sem1)

            @pl.when(jnp.logical_not(even))
            def _():
                start(i + 1, rows0, sem0)

        @pl.when(even)
        def _():
            pltpu.make_async_copy(y_hbm.at[cib.at[pl.ds(i * 2 * _CT, 2 * _CT)]],
                                  rows0, sem0).wait()
            compute(i, rows0)

        @pl.when(jnp.logical_not(even))
        def _():
            pltpu.make_async_copy(y_hbm.at[cib.at[pl.ds(i * 2 * _CT, 2 * _CT)]],
                                  rows1, sem1).wait()
            compute(i, rows1)
        return 0

    lax.fori_loop(0, _CN, step, 0)


def _sc_combine(Yf_i32, cidx, cgate):
    mesh = plsc.VectorSubcoreMesh(core_axis_name="c", subcore_axis_name="s")
    fn = pl.kernel(
        _combine_body,
        out_type=jax.ShapeDtypeStruct((G * S, DIM // 2), jnp.int32),
        mesh=mesh,
        scratch_types=[pltpu.VMEM((_CW * 2,), jnp.int32),
                       pltpu.VMEM((_CW * 2,), jnp.float32),
                       pltpu.VMEM((2 * _CT, DIM // 2), jnp.int32),
                       pltpu.VMEM((2 * _CT, DIM // 2), jnp.int32),
                       pltpu.VMEM((_CT, DIM // 2), jnp.int32),
                       pltpu.SemaphoreType.DMA,
                       pltpu.SemaphoreType.DMA],
        compiler_params=pltpu.CompilerParams(needs_layout_passes=False),
    )
    return fn(Yf_i32, cidx, cgate)


def _unpack_body(x_ref, o_ref):
    lo, hi = _unpack_pair(x_ref[...])
    o_ref[...] = jnp.concatenate([lo, hi], axis=-1)


def _unpack_out(o32):
    return pl.pallas_call(
        _unpack_body,
        grid=(G,),
        in_specs=[pl.BlockSpec((S, DIM // 2), lambda i: (i, 0))],
        out_specs=pl.BlockSpec((S, DIM), lambda i: (i, 0)),
        out_shape=jax.ShapeDtypeStruct((G * S, DIM), jnp.bfloat16),
    )(o32)


def kernel(inputs, wi_gate_0, wi_0, wo_0, router_gate):
    bf = inputs.dtype
    xf = inputs.reshape(G * S, DIM)
    logits, x32 = _router_logits(xf, router_gate)
    slot_token, cidx, cgate = _sc_route(logits)

    Xe_i32 = _sc_gather(x32, slot_token)
    Y_i32 = _expert_mlp(Xe_i32.reshape(E, M, DIM // 2), wi_gate_0.astype(bf),
                        wi_0.astype(bf), wo_0.astype(bf))
    o32 = _sc_combine(Y_i32.reshape(NSLOT, DIM // 2), cidx, cgate)
    out = _unpack_out(o32)
    return out.reshape(inputs.shape)


# speedup vs baseline: 1.1869x; 1.0040x over previous
import functools

import jax
import jax.numpy as jnp
from jax import lax
from jax.experimental import pallas as pl
from jax.experimental.pallas import tpu as pltpu
from jax.experimental.pallas import tpu_sc as plsc

G, S, DIM, E, TOPN = 10, 2048, 2560, 8, 2
INTER = 6912
C = 384
M = G * C
MC = 768
NM = M // MC
HB = 768
NH = INTER // HB
NSLOT = E * M


def _pack_pair(lo_bf, hi_bf):
    li = lax.bitcast_convert_type(lo_bf.astype(jnp.float32), jnp.int32)
    ri = lax.bitcast_convert_type(hi_bf.astype(jnp.float32), jnp.int32)
    return lax.shift_right_logical(li, 16) | (ri & jnp.int32(-65536))


def _unpack_pair(x32):
    lo = lax.bitcast_convert_type(lax.shift_left(x32, 16), jnp.float32)
    hi = lax.bitcast_convert_type(x32 & jnp.int32(-65536), jnp.float32)
    return lo.astype(jnp.bfloat16), hi.astype(jnp.bfloat16)


def _logits_body(x_ref, w_ref, o_ref, x32_ref):
    xb = x_ref[...]
    o_ref[...] = jnp.dot(xb.astype(jnp.float32), w_ref[...],
                         preferred_element_type=jnp.float32)
    x32_ref[...] = _pack_pair(xb[:, :DIM // 2], xb[:, DIM // 2:])


def _router_logits(xf, router_gate):
    wpad = jnp.zeros((DIM, 128), jnp.float32).at[:, :E].set(router_gate)
    out, x32 = pl.pallas_call(
        _logits_body,
        grid=(G,),
        in_specs=[pl.BlockSpec((S, DIM), lambda i: (i, 0)),
                  pl.BlockSpec((DIM, 128), lambda i: (0, 0))],
        out_specs=[pl.BlockSpec((S, 128), lambda i: (i, 0)),
                   pl.BlockSpec((S, DIM // 2), lambda i: (i, 0))],
        out_shape=[jax.ShapeDtypeStruct((G * S, 128), jnp.float32),
                   jax.ShapeDtypeStruct((G * S, DIM // 2), jnp.int32)],
    )(xf, wpad)
    return out[:, :E], x32


def _mlp_body(x_ref, wg_ref, wi_ref, wo_ref, o_ref, acc_ref, xl_ref, xr_ref):
    h = pl.program_id(2)

    @pl.when(h == 0)
    def _():
        xl, xr = _unpack_pair(x_ref[0])
        xl_ref[...] = xl
        xr_ref[...] = xr

    xl = xl_ref[...]
    xr = xr_ref[...]
    D2 = DIM // 2
    wg = wg_ref[0]
    wi = wi_ref[0]
    h1 = (jnp.dot(xl, wg[:D2], preferred_element_type=jnp.float32)
          + jnp.dot(xr, wg[D2:], preferred_element_type=jnp.float32))
    h0 = (jnp.dot(xl, wi[:D2], preferred_element_type=jnp.float32)
          + jnp.dot(xr, wi[D2:], preferred_element_type=jnp.float32))
    hh = jax.nn.silu(h1.astype(jnp.bfloat16)) * h0.astype(jnp.bfloat16)
    y = jnp.dot(hh, wo_ref[0], preferred_element_type=jnp.float32)

    @pl.when(h == 0)
    def _():
        acc_ref[...] = y

    @pl.when(h > 0)
    def _():
        acc_ref[...] += y

    @pl.when(h == NH - 1)
    def _():
        yb = acc_ref[...].astype(jnp.bfloat16)
        o_ref[0] = _pack_pair(yb[:, :D2], yb[:, D2:])


def _expert_mlp(Xe_i32, wg, wi, wo):
    return pl.pallas_call(
        _mlp_body,
        grid=(E, NM, NH),
        in_specs=[
            pl.BlockSpec((1, MC, DIM // 2), lambda e, m, h: (e, m, 0)),
            pl.BlockSpec((1, DIM, HB), lambda e, m, h: (e, 0, h)),
            pl.BlockSpec((1, DIM, HB), lambda e, m, h: (e, 0, h)),
            pl.BlockSpec((1, HB, DIM), lambda e, m, h: (e, h, 0)),
        ],
        out_specs=pl.BlockSpec((1, MC, DIM // 2), lambda e, m, h: (e, m, 0)),
        out_shape=jax.ShapeDtypeStruct((E, M, DIM // 2), jnp.int32),
        scratch_shapes=[pltpu.VMEM((MC, DIM), jnp.float32),
                        pltpu.VMEM((MC, DIM // 2), jnp.bfloat16),
                        pltpu.VMEM((MC, DIM // 2), jnp.bfloat16)],
        compiler_params=pltpu.CompilerParams(
            dimension_semantics=("parallel", "parallel", "arbitrary")),
    )(Xe_i32, wg, wi, wo)


_INFO = plsc.get_sparse_core_info()
NC, NS = _INFO.num_cores, _INFO.num_subcores
NW = NC * NS
TB = 16
NBLK = S // TB


def _wid():
    return lax.axis_index("s") * NC + lax.axis_index("c")


def _dyn_splat_i(vec, lane):
    dnums = lax.GatherDimensionNumbers(
        offset_dims=(), collapsed_slice_dims=(0,), start_index_map=(0,))
    idx = jnp.full((16, 1), lane, jnp.int32)
    return lax.gather(vec, idx, dnums, slice_sizes=(1,),
                      mode=lax.GatherScatterMode.PROMISE_IN_BOUNDS)


def _route_body(lg_hbm, stok_hbm, cidx_hbm, cgate_hbm,
                lg, stok, cbuf, gbuf):
    g = _wid()
    zi = jnp.zeros((16,), jnp.int32)

    @pl.when(g < G)
    def _():
        pltpu.sync_copy(lg_hbm.at[pl.ds(g * S * E, S * E)], lg)
        def _z(i, _):
            stok[pl.ds(i * 16, 16)] = zi
            return 0
        lax.fori_loop(0, (E * C) // 16, _z, 0)

        iot = lax.iota(jnp.int32, 16)

        def blk(b, cnt):
            t0 = b * TB
            vs = []
            for e in range(E):
                vs.append(plsc.load_gather(lg, [(t0 + iot) * E + e]))
            max1 = vs[0]
            for e in range(1, E):
                max1 = jnp.maximum(max1, vs[e])
            arg1 = zi + E
            for e in range(E - 1, -1, -1):
                arg1 = jnp.where(vs[e] == max1, e, arg1)
            max2 = None
            for e in range(E):
                m = jnp.where(arg1 == e, -jnp.inf, vs[e])
                max2 = m if max2 is None else jnp.maximum(max2, m)
            arg2 = zi + E
            for e in range(E - 1, -1, -1):
                arg2 = jnp.where(
                    (arg1 != e) & (jnp.where(arg1 == e, -jnp.inf, vs[e]) == max2),
                    e, arg2)
            bexp = jnp.exp(max2 - max1)
            g1 = 1.0 / (1.0 + bexp)
            g2 = bexp / (1.0 + bexp)

            pos1 = zi
            pos2 = zi
            ncnt = []
            for e in range(E):
                m1 = arg1 == e
                m2 = arg2 == e
                i1 = jnp.where(m1, 1, 0)
                i2 = jnp.where(m2, 1, 0)
                c1 = plsc.cumsum(i1)
                c2 = plsc.cumsum(i2)
                base = cnt[e]
                pos1 = jnp.where(m1, base + c1 - 1 + c2 - i2, pos1)
                pos2 = jnp.where(m2, base + c1 + c2 - 1, pos2)
                ncnt.append(base + _dyn_splat_i(c1, 15) + _dyn_splat_i(c2, 15))
            cnt = tuple(ncnt)
            ok1 = pos1 < C
            ok2 = pos2 < C
            tok = g * S + t0 + iot
            plsc.store_scatter(stok, [arg1 * C + pos1], tok, mask=ok1)
            plsc.store_scatter(stok, [arg2 * C + pos2], tok, mask=ok2)
            slot1 = (arg1 * G + g) * C + pos1
            slot2 = (arg2 * G + g) * C + pos2
            ai = 2 * (t0 + iot)
            plsc.store_scatter(cbuf, [ai], jnp.where(ok1, slot1, 0))
            plsc.store_scatter(cbuf, [ai + 1], jnp.where(ok2, slot2, 0))
            plsc.store_scatter(gbuf, [ai], jnp.where(ok1, g1, 0.0))
            plsc.store_scatter(gbuf, [ai + 1], jnp.where(ok2, g2, 0.0))
            return cnt

        zv = jnp.zeros((16,), jnp.int32)
        lax.fori_loop(0, NBLK, blk, tuple(zv for _ in range(E)),
                      unroll=False)

        for e in range(E):
            pltpu.sync_copy(stok.at[pl.ds(e * C, C)],
                            stok_hbm.at[pl.ds((e * G + g) * C, C)])
        pltpu.sync_copy(cbuf, cidx_hbm.at[pl.ds(g * S * TOPN, S * TOPN)])
        pltpu.sync_copy(gbuf, cgate_hbm.at[pl.ds(g * S * TOPN, S * TOPN)])


def _sc_route(logits8):
    mesh = plsc.VectorSubcoreMesh(core_axis_name="c", subcore_axis_name="s")
    fn = pl.kernel(
        _route_body,
        out_type=[jax.ShapeDtypeStruct((NSLOT,), jnp.int32),
                  jax.ShapeDtypeStruct((G * S * TOPN,), jnp.int32),
                  jax.ShapeDtypeStruct((G * S * TOPN,), jnp.float32)],
        mesh=mesh,
        scratch_types=[pltpu.VMEM((S * E,), jnp.float32),
                       pltpu.VMEM((E * C,), jnp.int32),
                       pltpu.VMEM((S * TOPN,), jnp.int32),
                       pltpu.VMEM((S * TOPN,), jnp.float32)],
        compiler_params=pltpu.CompilerParams(needs_layout_passes=False),
    )
    return fn(logits8.reshape(G * S * E))


_GK = 48
_GR = NSLOT // NW
_GN = _GR // _GK


def _gather_body(xf_hbm, st_hbm, xe_hbm, idxb, rows0, rows1, sem0, sem1):
    w = _wid()
    base = w * _GR
    pltpu.sync_copy(st_hbm.at[pl.ds(base, _GR)], idxb)

    def start(i, rows, sem):
        pltpu.async_copy(xf_hbm.at[idxb.at[pl.ds(i * _GK, _GK)]], rows, sem)

    start(0, rows0, sem0)

    def step(i, _):
        even = lax.rem(i, 2) == 0

        @pl.when(i + 1 < _GN)
        def _():
            @pl.when(even)
            def _():
                start(i + 1, rows1, sem1)

            @pl.when(jnp.logical_not(even))
            def _():
                start(i + 1, rows0, sem0)

        @pl.when(even)
        def _():
            pltpu.make_async_copy(xf_hbm.at[idxb.at[pl.ds(i * _GK, _GK)]],
                                  rows0, sem0).wait()
            pltpu.sync_copy(rows0, xe_hbm.at[pl.ds(base + i * _GK, _GK)])

        @pl.when(jnp.logical_not(even))
        def _():
            pltpu.make_async_copy(xf_hbm.at[idxb.at[pl.ds(i * _GK, _GK)]],
                                  rows1, sem1).wait()
            pltpu.sync_copy(rows1, xe_hbm.at[pl.ds(base + i * _GK, _GK)])
        return 0

    lax.fori_loop(0, _GN, step, 0)


def _sc_gather(xf_i32, slot_token):
    mesh = plsc.VectorSubcoreMesh(core_axis_name="c", subcore_axis_name="s")
    fn = pl.kernel(
        _gather_body,
        out_type=jax.ShapeDtypeStruct((NSLOT, DIM // 2), jnp.int32),
        mesh=mesh,
        scratch_types=[pltpu.VMEM((_GR,), jnp.int32),
                       pltpu.VMEM((_GK, DIM // 2), jnp.int32),
                       pltpu.VMEM((_GK, DIM // 2), jnp.int32),
                       pltpu.SemaphoreType.DMA,
                       pltpu.SemaphoreType.DMA],
        compiler_params=pltpu.CompilerParams(needs_layout_passes=False),
    )
    return fn(xf_i32, slot_token)


_CT = 16
_CW = (G * S) // NW
_CN = _CW // _CT


def _dyn_splat(vec, lane):
    dnums = lax.GatherDimensionNumbers(
        offset_dims=(), collapsed_slice_dims=(0,), start_index_map=(0,))
    idx = jnp.full((16, 1), lane, jnp.int32)
    return lax.gather(vec, idx, dnums, slice_sizes=(1,),
                      mode=lax.GatherScatterMode.PROMISE_IN_BOUNDS)


def _combine_body(y_hbm, ci_hbm, cg_hbm, out_hbm, cib, cgb, rows0, rows1,
                  obuf, sem0, sem1):
    w = _wid()
    tbase = w * _CW
    pltpu.sync_copy(ci_hbm.at[pl.ds(tbase * 2, _CW * 2)], cib)
    pltpu.sync_copy(cg_hbm.at[pl.ds(tbase * 2, _CW * 2)], cgb)

    def start(i, rows, sem):
        pltpu.async_copy(y_hbm.at[cib.at[pl.ds(i * 2 * _CT, 2 * _CT)]], rows, sem)

    start(0, rows0, sem0)

    def compute(i, rows):
        for t in range(_CT):
            gv = cgb[pl.ds(i * 2 * _CT + (2 * t // 16) * 16, 16)]
            ga = _dyn_splat(gv, (2 * t) % 16)
            gb = _dyn_splat(gv, (2 * t + 1) % 16)
            gab = plsc.pack(ga, ga, format=plsc.PackFormat.INTERLEAVED)
            gbb = plsc.pack(gb, gb, format=plsc.PackFormat.INTERLEAVED)

            def chunk(j, _):
                a = plsc.bitcast(rows[2 * t, pl.ds(j * 16, 16)], jnp.bfloat16)
                bb = plsc.bitcast(rows[2 * t + 1, pl.ds(j * 16, 16)],
                                  jnp.bfloat16)
                obuf[t, pl.ds(j * 16, 16)] = plsc.bitcast(a * gab + bb * gbb,
                                                          jnp.int32)
                return 0

            lax.fori_loop(0, DIM // 32, chunk, 0, unroll=8)
        pltpu.sync_copy(obuf, out_hbm.at[pl.ds(tbase + i * _CT, _CT)])

    def step(i, _):
        even = lax.rem(i, 2) == 0

        @pl.when(i + 1 < _CN)
        def _():
            @pl.when(even)
            def _():
                start(i + 1, rows1, sem1)

            @pl.when(jnp.logical_not(even))
            def _():
                start(i + 1, rows0, sem0)

        @pl.when(even)
        def _():
            pltpu.make_async_copy(y_hbm.at[cib.at[pl.ds(i * 2 * _CT, 2 * _CT)]],
                                  rows0, sem0).wait()
            compute(i, rows0)

        @pl.when(jnp.logical_not(even))
        def _():
            pltpu.make_async_copy(y_hbm.at[cib.at[pl.ds(i * 2 * _CT, 2 * _CT)]],
                                  rows1, sem1).wait()
            compute(i, rows1)
        return 0

    lax.fori_loop(0, _CN, step, 0)


def _sc_combine(Yf_i32, cidx, cgate):
    mesh = plsc.VectorSubcoreMesh(core_axis_name="c", subcore_axis_name="s")
    fn = pl.kernel(
        _combine_body,
        out_type=jax.ShapeDtypeStruct((G * S, DIM // 2), jnp.int32),
        mesh=mesh,
        scratch_types=[pltpu.VMEM((_CW * 2,), jnp.int32),
                       pltpu.VMEM((_CW * 2,), jnp.float32),
                       pltpu.VMEM((2 * _CT, DIM // 2), jnp.int32),
                       pltpu.VMEM((2 * _CT, DIM // 2), jnp.int32),
                       pltpu.VMEM((_CT, DIM // 2), jnp.int32),
                       pltpu.SemaphoreType.DMA,
                       pltpu.SemaphoreType.DMA],
        compiler_params=pltpu.CompilerParams(needs_layout_passes=False),
    )
    return fn(Yf_i32, cidx, cgate)


def _unpack_body(x_ref, o_ref):
    lo, hi = _unpack_pair(x_ref[...])
    o_ref[...] = jnp.concatenate([lo, hi], axis=-1)


def _unpack_out(o32):
    return pl.pallas_call(
        _unpack_body,
        grid=(G,),
        in_specs=[pl.BlockSpec((S, DIM // 2), lambda i: (i, 0))],
        out_specs=pl.BlockSpec((S, DIM), lambda i: (i, 0)),
        out_shape=jax.ShapeDtypeStruct((G * S, DIM), jnp.bfloat16),
    )(o32)


def kernel(inputs, wi_gate_0, wi_0, wo_0, router_gate):
    bf = inputs.dtype
    xf = inputs.reshape(G * S, DIM)
    logits, x32 = _router_logits(xf, router_gate)
    slot_token, cidx, cgate = _sc_route(logits)

    Xe_i32 = _sc_gather(x32, slot_token)
    Y_i32 = _expert_mlp(Xe_i32.reshape(E, M, DIM // 2), wi_gate_0.astype(bf),
                        wi_0.astype(bf), wo_0.astype(bf))
    o32 = _sc_combine(Y_i32.reshape(NSLOT, DIM // 2), cidx, cgate)
    out = _unpack_out(o32)
    return out.reshape(inputs.shape)
